# baseline (speedup 1.0000x reference)
import jax
import jax.numpy as jnp
from jax.experimental import pallas as pl
from jax.experimental.pallas import tpu as pltpu

_NUM_BINS = 64
_SIGMA = 0.5
_EPS = 1e-10
_P = 16384


def _unnorm_weights_t(ref):
    p = ref.shape[-1]
    v = jnp.clip(ref[0, 0], 0.0, 1.0)
    rowi = jax.lax.broadcasted_iota(jnp.int32, (_NUM_BINS, p), 0)
    rowf = rowi.astype(jnp.float32)
    d = v - rowf * (1.0 / (_NUM_BINS - 1))
    return jnp.exp(-2.0 * d * d)


def _accum_kernel(x_ref, y_ref, g_ref):
    nc = pl.program_id(1)

    @pl.when(nc == 0)
    def _():
        g_ref[...] = jnp.zeros_like(g_ref)

    ex = _unnorm_weights_t(x_ref)
    ey = _unnorm_weights_t(y_ref)
    sx = jnp.sum(ex, axis=0, keepdims=True)
    sy = jnp.sum(ey, axis=0, keepdims=True)
    a = ex * (1.0 / (sx + _EPS))
    b = ey * (1.0 / (sy + _EPS))
    g_ref[0] += jax.lax.dot_general(
        a, b, (((1,), (1,)), ((), ())), preferred_element_type=jnp.float32
    )


def _pixel_norm(x):
    centers = jnp.linspace(0.0, 1.0, _NUM_BINS, dtype=x.dtype)
    diff = x[:, :, None] - centers[None, None, :]
    w1 = jnp.exp(-0.5 * (diff / _SIGMA) ** 2)
    z = jnp.sum(w1, axis=-1, keepdims=True)
    return jax.lax.optimization_barrier(z)


def _marginal_hist(x, z):
    centers = jnp.linspace(0.0, 1.0, _NUM_BINS, dtype=x.dtype)
    diff = x[:, :, None] - centers[None, None, :]
    w = jnp.exp(-0.5 * (diff / _SIGMA) ** 2)
    w = w / (z + _EPS)
    return jnp.sum(w, axis=1)


def kernel(fixed, moving):
    b = fixed.shape[0]
    if fixed.shape[1] == 3:
        fixed = (0.299 * fixed[:, 0] + 0.587 * fixed[:, 1]
                 + 0.114 * fixed[:, 2])[:, None]
    if moving.shape[1] == 3:
        moving = (0.299 * moving[:, 0] + 0.587 * moving[:, 1]
                  + 0.114 * moving[:, 2])[:, None]
    n = fixed.size // b
    nc = n // _P
    xr = fixed.reshape(b, nc, 1, _P)
    yr = moving.reshape(b, nc, 1, _P)

    fx = jnp.clip(fixed.reshape(b, -1), 0.0, 1.0)
    mv = jnp.clip(moving.reshape(b, -1), 0.0, 1.0)
    zx = _pixel_norm(fx)
    zy = _pixel_norm(mv)

    in_spec = pl.BlockSpec((1, 1, 1, _P), lambda i, j: (i, j, 0, 0))
    joint = pl.pallas_call(
        _accum_kernel,
        out_shape=jax.ShapeDtypeStruct((b, _NUM_BINS, _NUM_BINS), jnp.float32),
        grid=(b, nc),
        in_specs=[in_spec, in_spec],
        out_specs=pl.BlockSpec((1, _NUM_BINS, _NUM_BINS), lambda i, j: (i, 0, 0)),
        compiler_params=pltpu.CompilerParams(
            dimension_semantics=("parallel", "arbitrary"),
        ),
        name="mi_gram_accum",
    )(xr, yr)

    hist_x = _marginal_hist(fx, zx)
    hist_x = hist_x / (jnp.sum(hist_x, axis=-1, keepdims=True) + _EPS)
    hist_y = _marginal_hist(mv, zy)
    hist_y = hist_y / (jnp.sum(hist_y, axis=-1, keepdims=True) + _EPS)
    joint = joint / (jnp.sum(joint, axis=(-1, -2), keepdims=True) + _EPS)

    def _ent(p):
        p = p + _EPS
        return -jnp.sum(p * jnp.log(p), axis=-1)

    mi = _ent(hist_x) + _ent(hist_y) - _ent(joint.reshape(b, -1))
    return -jnp.mean(mi)


# speedup vs baseline: 1.0400x; 1.0400x over previous
import jax
import jax.numpy as jnp
from jax.experimental import pallas as pl
from jax.experimental.pallas import tpu as pltpu

_NUM_BINS = 64
_SIGMA = 0.5
_EPS = 1e-10
_P = 16384


def _unnorm_weights_t(ref):
    p = ref.shape[-1]
    v = jnp.clip(ref[0, 0], 0.0, 1.0)
    rowi = jax.lax.broadcasted_iota(jnp.int32, (_NUM_BINS, p), 0)
    rowf = rowi.astype(jnp.float32)
    d = v - rowf * (1.0 / (_NUM_BINS - 1))
    return jnp.exp(-2.0 * d * d)


def _accum_kernel(x_ref, y_ref, zx_ref, zy_ref, g_ref):
    nc = pl.program_id(1)

    @pl.when(nc == 0)
    def _():
        g_ref[...] = jnp.zeros_like(g_ref)

    ex = _unnorm_weights_t(x_ref)
    ey = _unnorm_weights_t(y_ref)
    a = ex * (1.0 / (zx_ref[0, 0] + _EPS))
    b = ey * (1.0 / (zy_ref[0, 0] + _EPS))
    g_ref[0] += jax.lax.dot_general(
        a, b, (((1,), (1,)), ((), ())), preferred_element_type=jnp.float32
    )


def _pixel_norm(x):
    centers = jnp.linspace(0.0, 1.0, _NUM_BINS, dtype=x.dtype)
    diff = x[:, :, None] - centers[None, None, :]
    w1 = jnp.exp(-0.5 * (diff / _SIGMA) ** 2)
    z = jnp.sum(w1, axis=-1, keepdims=True)
    return jax.lax.optimization_barrier(z)


def _marginal_hist(x, z):
    centers = jnp.linspace(0.0, 1.0, _NUM_BINS, dtype=x.dtype)
    diff = x[:, :, None] - centers[None, None, :]
    w = jnp.exp(-0.5 * (diff / _SIGMA) ** 2)
    w = w / (z + _EPS)
    return jnp.sum(w, axis=1)


def kernel(fixed, moving):
    b = fixed.shape[0]
    if fixed.shape[1] == 3:
        fixed = (0.299 * fixed[:, 0] + 0.587 * fixed[:, 1]
                 + 0.114 * fixed[:, 2])[:, None]
    if moving.shape[1] == 3:
        moving = (0.299 * moving[:, 0] + 0.587 * moving[:, 1]
                  + 0.114 * moving[:, 2])[:, None]
    n = fixed.size // b
    nc = n // _P
    xr = fixed.reshape(b, nc, 1, _P)
    yr = moving.reshape(b, nc, 1, _P)

    fx = jnp.clip(fixed.reshape(b, -1), 0.0, 1.0)
    mv = jnp.clip(moving.reshape(b, -1), 0.0, 1.0)
    zx = _pixel_norm(fx)
    zy = _pixel_norm(mv)
    zxr = zx.reshape(b, nc, 1, _P)
    zyr = zy.reshape(b, nc, 1, _P)

    in_spec = pl.BlockSpec((1, 1, 1, _P), lambda i, j: (i, j, 0, 0))
    joint = pl.pallas_call(
        _accum_kernel,
        out_shape=jax.ShapeDtypeStruct((b, _NUM_BINS, _NUM_BINS), jnp.float32),
        grid=(b, nc),
        in_specs=[in_spec, in_spec, in_spec, in_spec],
        out_specs=pl.BlockSpec((1, _NUM_BINS, _NUM_BINS), lambda i, j: (i, 0, 0)),
        compiler_params=pltpu.CompilerParams(
            dimension_semantics=("parallel", "arbitrary"),
        ),
        name="mi_gram_accum",
    )(xr, yr, zxr, zyr)

    hist_x = _marginal_hist(fx, zx)
    hist_x = hist_x / (jnp.sum(hist_x, axis=-1, keepdims=True) + _EPS)
    hist_y = _marginal_hist(mv, zy)
    hist_y = hist_y / (jnp.sum(hist_y, axis=-1, keepdims=True) + _EPS)
    joint = joint / (jnp.sum(joint, axis=(-1, -2), keepdims=True) + _EPS)

    def _ent(p):
        p = p + _EPS
        return -jnp.sum(p * jnp.log(p), axis=-1)

    mi = _ent(hist_x) + _ent(hist_y) - _ent(joint.reshape(b, -1))
    return -jnp.mean(mi)
